# baseline (speedup 1.0000x reference)
import functools

import jax
import jax.numpy as jnp
from jax import lax
from jax.experimental import pallas as pl
from jax.experimental.pallas import tpu as pltpu
from jax.experimental.pallas import tpu_sc as plsc

B = 16
N = 4096
NQ = 1024
K = 32
EPS = 1e-5

_INTERPRET = False


def _fma_sq_acc(a, s):
    c = a * 4097.0
    dh = c - (c - a)
    dl = a - dh
    p = a * a
    e = ((dh * dh - p) + 2.0 * dh * dl) + dl * dl
    t = s + p
    z2 = t - s
    err2 = (s - (t - z2)) + (p - z2)
    return t + (err2 + e)


def _fps_body(xyz_ref, nxyz_ref, dist_ref):
    x = xyz_ref[:, 0, :]
    y = xyz_ref[:, 1, :]
    z = xyz_ref[:, 2, :]
    lane = lax.broadcasted_iota(jnp.int32, (B, N), 1)
    lane3 = lax.broadcasted_iota(jnp.int32, (B, 3, NQ), 2)
    dist_ref[...] = jnp.full((B, N), jnp.inf, jnp.float32)

    def body(i, far):
        m = lane == far
        fx = jnp.sum(jnp.where(m, x, 0.0), axis=1, keepdims=True)
        fy = jnp.sum(jnp.where(m, y, 0.0), axis=1, keepdims=True)
        fz = jnp.sum(jnp.where(m, z, 0.0), axis=1, keepdims=True)
        fcat = jnp.concatenate([fx[:, None, :], fy[:, None, :], fz[:, None, :]],
                               axis=1)
        nxyz_ref[...] = jnp.where(lane3 == i, fcat, nxyz_ref[...])
        dx = x - fx
        dy = y - fy
        dz = z - fz
        d = _fma_sq_acc(dz, _fma_sq_acc(dy, dx * dx))
        nd = jnp.minimum(dist_ref[...], d)
        dist_ref[...] = nd
        return jnp.argmax(nd, axis=1).astype(jnp.int32)[:, None]

    lax.fori_loop(0, NQ, body, jnp.zeros((B, 1), jnp.int32))


def _fps(xyz):
    return pl.pallas_call(
        _fps_body,
        out_shape=jax.ShapeDtypeStruct((B, 3, NQ), jnp.float32),
        scratch_shapes=[pltpu.VMEM((B, N), jnp.float32)],
        interpret=_INTERPRET,
    )(xyz)


_QT = 128


def _knn_body(nxyz_ref, xyz_ref, idx_ref, dist_ref):
    q = jnp.transpose(nxyz_ref[0])
    kk = xyz_ref[0]
    cross = lax.dot_general(q, kk, (((1,), (0,)), ((), ())),
                            preferred_element_type=jnp.float32)
    qsq = jnp.sum(q * q, axis=1, keepdims=True)
    ksq = jnp.sum(kk * kk, axis=0, keepdims=True)
    dist_ref[...] = (-2.0 * cross + qsq) + ksq
    lane = lax.broadcasted_iota(jnp.int32, (_QT, N), 1)
    krow = lax.broadcasted_iota(jnp.int32, (1, K, _QT), 1)

    def body(k, am_prev):
        dd = jnp.where(lane == am_prev, jnp.inf, dist_ref[...])
        dist_ref[...] = dd
        mval = jnp.min(dd, axis=1, keepdims=True)
        cand = jnp.where(dd == mval, lane, N)
        am = jnp.min(cand, axis=1, keepdims=True)
        idx_ref[...] = jnp.where(krow == k, am[:, 0][None, None, :], idx_ref[...])
        return am

    lax.fori_loop(0, K, body, jnp.full((_QT, 1), -1, jnp.int32))


def _knn(new_xyz, xyz):
    return pl.pallas_call(
        _knn_body,
        grid=(B, NQ // _QT),
        in_specs=[
            pl.BlockSpec((1, 3, _QT), lambda b, t: (b, 0, t)),
            pl.BlockSpec((1, 3, N), lambda b, t: (b, 0, 0)),
        ],
        out_specs=pl.BlockSpec((1, K, _QT), lambda b, t: (b, 0, t)),
        out_shape=jax.ShapeDtypeStruct((B, K, NQ), jnp.int32),
        scratch_shapes=[pltpu.VMEM((_QT, N), jnp.float32)],
        interpret=_INTERPRET,
    )(new_xyz, xyz)


def _prep_body(xyzT_ref, ptsT_ref, out_ref):
    xt = xyzT_ref[0]
    pad = jnp.zeros((N, 13), jnp.float32)
    out_ref[0, :, pl.ds(0, 16)] = jnp.concatenate([xt, pad], axis=1)
    out_ref[0, :, pl.ds(16, 32)] = ptsT_ref[0]


def _prep(xyzT, ptsT):
    return pl.pallas_call(
        _prep_body,
        grid=(B,),
        in_specs=[
            pl.BlockSpec((1, N, 3), lambda b: (b, 0, 0)),
            pl.BlockSpec((1, N, 32), lambda b: (b, 0, 0)),
        ],
        out_specs=pl.BlockSpec((1, N, 128), lambda b: (b, 0, 0)),
        out_shape=jax.ShapeDtypeStruct((B, N, 128), jnp.float32),
        interpret=_INTERPRET,
    )(xyzT, ptsT)


_P = B * NQ * K
_NW = 32
_CH = 512


def _sc_gather(table, gidx):
    mesh = plsc.VectorSubcoreMesh(core_axis_name="c", subcore_axis_name="s")
    per_w = _P // _NW

    @functools.partial(
        pl.kernel,
        mesh=mesh,
        out_type=jax.ShapeDtypeStruct((_P, 128), jnp.float32),
        scratch_types=[
            pltpu.VMEM((_CH,), jnp.int32),
            pltpu.VMEM((_CH, 128), jnp.float32),
            pltpu.SemaphoreType.DMA,
        ],
    )
    def k(table_hbm, idx_hbm, out_hbm, idx_v, rows_v, sem):
        wid = lax.axis_index("s") * 2 + lax.axis_index("c")
        base = wid * per_w

        def body(j, _):
            off = base + j * _CH
            pltpu.sync_copy(idx_hbm.at[pl.ds(off, _CH)], idx_v)
            pltpu.async_copy(table_hbm.at[idx_v], rows_v, sem).wait()
            pltpu.sync_copy(rows_v, out_hbm.at[pl.ds(off, _CH)])
            return 0

        lax.fori_loop(0, per_w // _CH, body, 0)

    return k(table, gidx)


_NT = 256
_G = B * NQ // _NT
_ROWS = _NT * K


def _y0(cg_ref, nxT_ref, W0pT_ref, b0_ref):
    a = cg_ref[0][:, :, :48]
    q = nxT_ref[0]
    qpad = jnp.concatenate([q, jnp.zeros((_NT, 45), jnp.float32)], axis=1)
    a = a - qpad[:, None, :]
    y = lax.dot_general(a.reshape(_ROWS, 48).astype(jnp.bfloat16), W0pT_ref[...],
                        (((1,), (0,)), ((), ())),
                        preferred_element_type=jnp.float32)
    return y + b0_ref[...]


def _bnrelu(y, s_ref, t_ref):
    return jnp.maximum(y * s_ref[...] + t_ref[...], 0.0)


def _mm(z, w_ref, b_ref):
    y = lax.dot_general(z.astype(jnp.bfloat16), w_ref[...], (((1,), (1,)), ((), ())),
                        preferred_element_type=jnp.float32)
    return y + b_ref[...]


def _stats_accum(y2d, out_ref):
    s = jnp.sum(y2d, axis=0, keepdims=True)
    ss = jnp.sum(y2d * y2d, axis=0, keepdims=True)
    both = jnp.concatenate([s, ss], axis=0)

    @pl.when(pl.program_id(0) == 0)
    def _():
        out_ref[...] = jnp.zeros_like(out_ref)

    out_ref[...] += both


_cg_spec = pl.BlockSpec((1, _NT, K, 128), lambda g: (g // 4, g % 4, 0, 0))
_nx_spec = pl.BlockSpec((1, _NT, 3), lambda g: (g // 4, g % 4, 0))


def _vec_spec(c):
    return pl.BlockSpec((1, c), lambda g: (0, 0))


def _w_spec(o, c):
    return pl.BlockSpec((o, c), lambda g: (0, 0))


def _pass1_body(cg_ref, nxT_ref, W0pT_ref, b0_ref, out_ref):
    y0 = _y0(cg_ref, nxT_ref, W0pT_ref, b0_ref)
    _stats_accum(y0, out_ref)


def _pass2_body(cg_ref, nxT_ref, W0pT_ref, b0_ref, s0_ref, t0_ref, W1_ref,
                b1_ref, out_ref):
    z0 = _bnrelu(_y0(cg_ref, nxT_ref, W0pT_ref, b0_ref), s0_ref, t0_ref)
    y1 = _mm(z0, W1_ref, b1_ref)
    _stats_accum(y1, out_ref)


def _pass3_body(cg_ref, nxT_ref, W0pT_ref, b0_ref, s0_ref, t0_ref, W1_ref,
                b1_ref, s1_ref, t1_ref, W2_ref, b2_ref, out_ref):
    z0 = _bnrelu(_y0(cg_ref, nxT_ref, W0pT_ref, b0_ref), s0_ref, t0_ref)
    z1 = _bnrelu(_mm(z0, W1_ref, b1_ref), s1_ref, t1_ref)
    y2 = _mm(z1, W2_ref, b2_ref)
    _stats_accum(y2, out_ref)


def _pass4_body(cg_ref, nxT_ref, W0pT_ref, b0_ref, s0_ref, t0_ref, W1_ref,
                b1_ref, s1_ref, t1_ref, W2_ref, b2_ref, s2_ref, t2_ref, out_ref):
    z0 = _bnrelu(_y0(cg_ref, nxT_ref, W0pT_ref, b0_ref), s0_ref, t0_ref)
    z1 = _bnrelu(_mm(z0, W1_ref, b1_ref), s1_ref, t1_ref)
    z2 = _bnrelu(_mm(z1, W2_ref, b2_ref), s2_ref, t2_ref)
    out_ref[0] = jnp.max(z2.reshape(_NT, K, 64), axis=1)


def _stats_call(body, extra_specs, cout, args):
    return pl.pallas_call(
        body,
        grid=(_G,),
        in_specs=[_cg_spec, _nx_spec, _w_spec(48, 32), _vec_spec(32)] + extra_specs,
        out_specs=pl.BlockSpec((2, cout), lambda g: (0, 0)),
        out_shape=jax.ShapeDtypeStruct((2, cout), jnp.float32),
        interpret=_INTERPRET,
    )(*args)


def _finalize(stats, gamma, beta):
    mean = stats[0] / _P
    var = stats[1] / _P - mean * mean
    s = gamma / jnp.sqrt(var + EPS)
    t = beta - mean * s
    return s[None, :], t[None, :]


def kernel(xyz, points, W0, b0, g0, be0, W1, b1, g1, be1, W2, b2, g2, be2):
    xyzT = jnp.transpose(xyz, (0, 2, 1))
    ptsT = jnp.transpose(points, (0, 2, 1))
    W0pT = jnp.concatenate(
        [jnp.transpose(W0[:, :3]), jnp.zeros((13, 32), jnp.float32),
         jnp.transpose(W0[:, 3:])], axis=0).astype(jnp.bfloat16)
    W1b = W1.astype(jnp.bfloat16)
    W2b = W2.astype(jnp.bfloat16)

    new_xyz = _fps(xyz)
    idx = _knn(new_xyz, xyz)
    table = _prep(xyzT, ptsT)

    gidx = (jnp.transpose(idx, (0, 2, 1))
            + (jnp.arange(B, dtype=jnp.int32) * N)[:, None, None]).reshape(-1)
    cg = _sc_gather(table.reshape(B * N, 128), gidx)
    cg4 = cg.reshape(B, NQ, K, 128)
    nxT = jnp.transpose(new_xyz, (0, 2, 1))

    b0r, b1r, b2r = b0[None, :], b1[None, :], b2[None, :]
    st0 = _stats_call(_pass1_body, [], 32, (cg4, nxT, W0pT, b0r))
    s0, t0 = _finalize(st0, g0, be0)
    st1 = _stats_call(_pass2_body,
                      [_vec_spec(32), _vec_spec(32), _w_spec(32, 32),
                       _vec_spec(32)], 32,
                      (cg4, nxT, W0pT, b0r, s0, t0, W1b, b1r))
    s1, t1 = _finalize(st1, g1, be1)
    st2 = _stats_call(_pass3_body,
                      [_vec_spec(32), _vec_spec(32), _w_spec(32, 32),
                       _vec_spec(32), _vec_spec(32), _vec_spec(32),
                       _w_spec(64, 32), _vec_spec(64)], 64,
                      (cg4, nxT, W0pT, b0r, s0, t0, W1b, b1r, s1, t1, W2b, b2r))
    s2, t2 = _finalize(st2, g2, be2)

    out = pl.pallas_call(
        _pass4_body,
        grid=(_G,),
        in_specs=[_cg_spec, _nx_spec, _w_spec(48, 32), _vec_spec(32),
                  _vec_spec(32), _vec_spec(32), _w_spec(32, 32), _vec_spec(32),
                  _vec_spec(32), _vec_spec(32), _w_spec(64, 32), _vec_spec(64),
                  _vec_spec(64), _vec_spec(64)],
        out_specs=pl.BlockSpec((1, _NT, 64), lambda g: (g // 4, g % 4, 0)),
        out_shape=jax.ShapeDtypeStruct((B, NQ, 64), jnp.float32),
        interpret=_INTERPRET,
    )(cg4, nxT, W0pT, b0r, s0, t0, W1b, b1r, s1, t1, W2b, b2r, s2, t2)

    return (new_xyz, jnp.transpose(out, (0, 2, 1)))


# speedup vs baseline: 1.1512x; 1.1512x over previous
import functools

import jax
import jax.numpy as jnp
from jax import lax
from jax.experimental import pallas as pl
from jax.experimental.pallas import tpu as pltpu
from jax.experimental.pallas import tpu_sc as plsc

B = 16
N = 4096
NQ = 1024
K = 32
EPS = 1e-5

_INTERPRET = False


def _fma_sq_acc(a, s):
    c = a * 4097.0
    dh = c - (c - a)
    dl = a - dh
    p = a * a
    e = ((dh * dh - p) + 2.0 * dh * dl) + dl * dl
    t = s + p
    z2 = t - s
    err2 = (s - (t - z2)) + (p - z2)
    return t + (err2 + e)


def _fps_body(xyz_ref, nxyz_ref, dist_ref):
    x = xyz_ref[:, 0, :]
    y = xyz_ref[:, 1, :]
    z = xyz_ref[:, 2, :]
    lane = lax.broadcasted_iota(jnp.int32, (B, N), 1)
    lane3 = lax.broadcasted_iota(jnp.int32, (B, 3, NQ), 2)
    dist_ref[...] = jnp.full((B, N), jnp.inf, jnp.float32)

    def body(i, far):
        m = lane == far
        fx = jnp.sum(jnp.where(m, x, 0.0), axis=1, keepdims=True)
        fy = jnp.sum(jnp.where(m, y, 0.0), axis=1, keepdims=True)
        fz = jnp.sum(jnp.where(m, z, 0.0), axis=1, keepdims=True)
        fcat = jnp.concatenate([fx[:, None, :], fy[:, None, :], fz[:, None, :]],
                               axis=1)
        nxyz_ref[...] = jnp.where(lane3 == i, fcat, nxyz_ref[...])
        dx = x - fx
        dy = y - fy
        dz = z - fz
        d = _fma_sq_acc(dz, _fma_sq_acc(dy, dx * dx))
        nd = jnp.minimum(dist_ref[...], d)
        dist_ref[...] = nd
        return jnp.argmax(nd, axis=1).astype(jnp.int32)[:, None]

    lax.fori_loop(0, NQ, body, jnp.zeros((B, 1), jnp.int32))


def _fps(xyz):
    return pl.pallas_call(
        _fps_body,
        out_shape=jax.ShapeDtypeStruct((B, 3, NQ), jnp.float32),
        scratch_shapes=[pltpu.VMEM((B, N), jnp.float32)],
        interpret=_INTERPRET,
    )(xyz)


_QT = 256


def _knn_body(nxyz_ref, xyz_ref, idx_ref, dist_ref):
    q = jnp.transpose(nxyz_ref[0])
    kk = xyz_ref[0]
    cross = lax.dot_general(q, kk, (((1,), (0,)), ((), ())),
                            preferred_element_type=jnp.float32)
    qsq = jnp.sum(q * q, axis=1, keepdims=True)
    ksq = jnp.sum(kk * kk, axis=0, keepdims=True)
    dist_ref[...] = (-2.0 * cross + qsq) + ksq
    lane = lax.broadcasted_iota(jnp.int32, (_QT, N), 1)
    krow = lax.broadcasted_iota(jnp.int32, (1, K, _QT), 1)

    def body(k, _):
        dd = dist_ref[...]
        am = jnp.argmin(dd, axis=1).astype(jnp.int32)
        idx_ref[...] = jnp.where(krow == k, am[None, None, :], idx_ref[...])
        dist_ref[...] = jnp.where(lane == am[:, None], jnp.inf, dd)
        return 0

    lax.fori_loop(0, K, body, 0)


def _knn(new_xyz, xyz):
    return pl.pallas_call(
        _knn_body,
        grid=(B, NQ // _QT),
        in_specs=[
            pl.BlockSpec((1, 3, _QT), lambda b, t: (b, 0, t)),
            pl.BlockSpec((1, 3, N), lambda b, t: (b, 0, 0)),
        ],
        out_specs=pl.BlockSpec((1, K, _QT), lambda b, t: (b, 0, t)),
        out_shape=jax.ShapeDtypeStruct((B, K, NQ), jnp.int32),
        scratch_shapes=[pltpu.VMEM((_QT, N), jnp.float32)],
        interpret=_INTERPRET,
    )(new_xyz, xyz)


def _prep_body(xyzT_ref, ptsT_ref, out_ref):
    xt = xyzT_ref[0]
    pad = jnp.zeros((N, 13), jnp.float32)
    out_ref[0, :, pl.ds(0, 16)] = jnp.concatenate([xt, pad], axis=1)
    out_ref[0, :, pl.ds(16, 32)] = ptsT_ref[0]


def _prep(xyzT, ptsT):
    return pl.pallas_call(
        _prep_body,
        grid=(B,),
        in_specs=[
            pl.BlockSpec((1, N, 3), lambda b: (b, 0, 0)),
            pl.BlockSpec((1, N, 32), lambda b: (b, 0, 0)),
        ],
        out_specs=pl.BlockSpec((1, N, 128), lambda b: (b, 0, 0)),
        out_shape=jax.ShapeDtypeStruct((B, N, 128), jnp.float32),
        interpret=_INTERPRET,
    )(xyzT, ptsT)


_P = B * NQ * K
_NW = 32
_CH = 512


def _sc_gather(table, gidx):
    mesh = plsc.VectorSubcoreMesh(core_axis_name="c", subcore_axis_name="s")
    per_w = _P // _NW

    @functools.partial(
        pl.kernel,
        mesh=mesh,
        out_type=jax.ShapeDtypeStruct((_P, 128), jnp.float32),
        scratch_types=[
            pltpu.VMEM((_CH,), jnp.int32),
            pltpu.VMEM((_CH, 128), jnp.float32),
            pltpu.SemaphoreType.DMA,
        ],
    )
    def k(table_hbm, idx_hbm, out_hbm, idx_v, rows_v, sem):
        wid = lax.axis_index("s") * 2 + lax.axis_index("c")
        base = wid * per_w

        def body(j, _):
            off = base + j * _CH
            pltpu.sync_copy(idx_hbm.at[pl.ds(off, _CH)], idx_v)
            pltpu.async_copy(table_hbm.at[idx_v], rows_v, sem).wait()
            pltpu.sync_copy(rows_v, out_hbm.at[pl.ds(off, _CH)])
            return 0

        lax.fori_loop(0, per_w // _CH, body, 0)

    return k(table, gidx)


_NT = 256
_G = B * NQ // _NT
_ROWS = _NT * K


def _y0(cg_ref, nxT_ref, W0pT_ref, b0_ref):
    a = cg_ref[0][:, :, :48]
    q = nxT_ref[0]
    qpad = jnp.concatenate([q, jnp.zeros((_NT, 45), jnp.float32)], axis=1)
    a = a - qpad[:, None, :]
    y = lax.dot_general(a.reshape(_ROWS, 48).astype(jnp.bfloat16), W0pT_ref[...],
                        (((1,), (0,)), ((), ())),
                        preferred_element_type=jnp.float32)
    return y + b0_ref[...]


def _bnrelu(y, s_ref, t_ref):
    return jnp.maximum(y * s_ref[...] + t_ref[...], 0.0)


def _mm(z, w_ref, b_ref):
    y = lax.dot_general(z.astype(jnp.bfloat16), w_ref[...], (((1,), (1,)), ((), ())),
                        preferred_element_type=jnp.float32)
    return y + b_ref[...]


def _stats_accum(y2d, out_ref):
    s = jnp.sum(y2d, axis=0, keepdims=True)
    ss = jnp.sum(y2d * y2d, axis=0, keepdims=True)
    both = jnp.concatenate([s, ss], axis=0)

    @pl.when(pl.program_id(0) == 0)
    def _():
        out_ref[...] = jnp.zeros_like(out_ref)

    out_ref[...] += both


_cg_spec = pl.BlockSpec((1, _NT, K, 128), lambda g: (g // 4, g % 4, 0, 0))
_nx_spec = pl.BlockSpec((1, _NT, 3), lambda g: (g // 4, g % 4, 0))


def _vec_spec(c):
    return pl.BlockSpec((1, c), lambda g: (0, 0))


def _w_spec(o, c):
    return pl.BlockSpec((o, c), lambda g: (0, 0))


def _pass1_body(cg_ref, nxT_ref, W0pT_ref, b0_ref, out_ref):
    y0 = _y0(cg_ref, nxT_ref, W0pT_ref, b0_ref)
    _stats_accum(y0, out_ref)


def _pass2_body(cg_ref, nxT_ref, W0pT_ref, b0_ref, s0_ref, t0_ref, W1_ref,
                b1_ref, out_ref):
    z0 = _bnrelu(_y0(cg_ref, nxT_ref, W0pT_ref, b0_ref), s0_ref, t0_ref)
    y1 = _mm(z0, W1_ref, b1_ref)
    _stats_accum(y1, out_ref)


def _pass3_body(cg_ref, nxT_ref, W0pT_ref, b0_ref, s0_ref, t0_ref, W1_ref,
                b1_ref, s1_ref, t1_ref, W2_ref, b2_ref, out_ref):
    z0 = _bnrelu(_y0(cg_ref, nxT_ref, W0pT_ref, b0_ref), s0_ref, t0_ref)
    z1 = _bnrelu(_mm(z0, W1_ref, b1_ref), s1_ref, t1_ref)
    y2 = _mm(z1, W2_ref, b2_ref)
    _stats_accum(y2, out_ref)


def _pass4_body(cg_ref, nxT_ref, W0pT_ref, b0_ref, s0_ref, t0_ref, W1_ref,
                b1_ref, s1_ref, t1_ref, W2_ref, b2_ref, s2_ref, t2_ref, out_ref):
    z0 = _bnrelu(_y0(cg_ref, nxT_ref, W0pT_ref, b0_ref), s0_ref, t0_ref)
    z1 = _bnrelu(_mm(z0, W1_ref, b1_ref), s1_ref, t1_ref)
    z2 = _bnrelu(_mm(z1, W2_ref, b2_ref), s2_ref, t2_ref)
    out_ref[0] = jnp.max(z2.reshape(_NT, K, 64), axis=1)


def _stats_call(body, extra_specs, cout, args):
    return pl.pallas_call(
        body,
        grid=(_G,),
        in_specs=[_cg_spec, _nx_spec, _w_spec(48, 32), _vec_spec(32)] + extra_specs,
        out_specs=pl.BlockSpec((2, cout), lambda g: (0, 0)),
        out_shape=jax.ShapeDtypeStruct((2, cout), jnp.float32),
        interpret=_INTERPRET,
    )(*args)


def _finalize(stats, gamma, beta):
    mean = stats[0] / _P
    var = stats[1] / _P - mean * mean
    s = gamma / jnp.sqrt(var + EPS)
    t = beta - mean * s
    return s[None, :], t[None, :]


def kernel(xyz, points, W0, b0, g0, be0, W1, b1, g1, be1, W2, b2, g2, be2):
    xyzT = jnp.transpose(xyz, (0, 2, 1))
    ptsT = jnp.transpose(points, (0, 2, 1))
    W0pT = jnp.concatenate(
        [jnp.transpose(W0[:, :3]), jnp.zeros((13, 32), jnp.float32),
         jnp.transpose(W0[:, 3:])], axis=0).astype(jnp.bfloat16)
    W1b = W1.astype(jnp.bfloat16)
    W2b = W2.astype(jnp.bfloat16)

    new_xyz = _fps(xyz)
    idx = _knn(new_xyz, xyz)
    table = _prep(xyzT, ptsT)

    gidx = (jnp.transpose(idx, (0, 2, 1))
            + (jnp.arange(B, dtype=jnp.int32) * N)[:, None, None]).reshape(-1)
    cg = _sc_gather(table.reshape(B * N, 128), gidx)
    cg4 = cg.reshape(B, NQ, K, 128)
    nxT = jnp.transpose(new_xyz, (0, 2, 1))

    b0r, b1r, b2r = b0[None, :], b1[None, :], b2[None, :]
    st0 = _stats_call(_pass1_body, [], 32, (cg4, nxT, W0pT, b0r))
    s0, t0 = _finalize(st0, g0, be0)
    st1 = _stats_call(_pass2_body,
                      [_vec_spec(32), _vec_spec(32), _w_spec(32, 32),
                       _vec_spec(32)], 32,
                      (cg4, nxT, W0pT, b0r, s0, t0, W1b, b1r))
    s1, t1 = _finalize(st1, g1, be1)
    st2 = _stats_call(_pass3_body,
                      [_vec_spec(32), _vec_spec(32), _w_spec(32, 32),
                       _vec_spec(32), _vec_spec(32), _vec_spec(32),
                       _w_spec(64, 32), _vec_spec(64)], 64,
                      (cg4, nxT, W0pT, b0r, s0, t0, W1b, b1r, s1, t1, W2b, b2r))
    s2, t2 = _finalize(st2, g2, be2)

    out = pl.pallas_call(
        _pass4_body,
        grid=(_G,),
        in_specs=[_cg_spec, _nx_spec, _w_spec(48, 32), _vec_spec(32),
                  _vec_spec(32), _vec_spec(32), _w_spec(32, 32), _vec_spec(32),
                  _vec_spec(32), _vec_spec(32), _w_spec(64, 32), _vec_spec(64),
                  _vec_spec(64), _vec_spec(64)],
        out_specs=pl.BlockSpec((1, _NT, 64), lambda g: (g // 4, g % 4, 0)),
        out_shape=jax.ShapeDtypeStruct((B, NQ, 64), jnp.float32),
        interpret=_INTERPRET,
    )(cg4, nxT, W0pT, b0r, s0, t0, W1b, b1r, s1, t1, W2b, b2r, s2, t2)

    return (new_xyz, jnp.transpose(out, (0, 2, 1)))


# speedup vs baseline: 1.2033x; 1.0453x over previous
import functools

import jax
import jax.numpy as jnp
from jax import lax
from jax.experimental import pallas as pl
from jax.experimental.pallas import tpu as pltpu
from jax.experimental.pallas import tpu_sc as plsc

B = 16
N = 4096
NQ = 1024
K = 32
EPS = 1e-5

_INTERPRET = False


def _fma_sq_acc(a, s):
    c = a * 4097.0
    dh = c - (c - a)
    dl = a - dh
    p = a * a
    e = ((dh * dh - p) + 2.0 * dh * dl) + dl * dl
    t = s + p
    z2 = t - s
    err2 = (s - (t - z2)) + (p - z2)
    return t + (err2 + e)


def _fps_body(xyz_ref, nxyz_ref, dist_ref):
    x = xyz_ref[:, 0, :]
    y = xyz_ref[:, 1, :]
    z = xyz_ref[:, 2, :]
    lane = lax.broadcasted_iota(jnp.int32, (B, N), 1)
    lane3 = lax.broadcasted_iota(jnp.int32, (B, 3, NQ), 2)
    dist_ref[...] = jnp.full((B, N), jnp.inf, jnp.float32)

    def body(i, far):
        m = lane == far
        fx = jnp.sum(jnp.where(m, x, 0.0), axis=1, keepdims=True)
        fy = jnp.sum(jnp.where(m, y, 0.0), axis=1, keepdims=True)
        fz = jnp.sum(jnp.where(m, z, 0.0), axis=1, keepdims=True)
        fcat = jnp.concatenate([fx[:, None, :], fy[:, None, :], fz[:, None, :]],
                               axis=1)
        nxyz_ref[...] = jnp.where(lane3 == i, fcat, nxyz_ref[...])
        dx = x - fx
        dy = y - fy
        dz = z - fz
        d = _fma_sq_acc(dz, _fma_sq_acc(dy, dx * dx))
        nd = jnp.minimum(dist_ref[...], d)
        dist_ref[...] = nd
        return jnp.argmax(nd, axis=1).astype(jnp.int32)[:, None]

    lax.fori_loop(0, NQ, body, jnp.zeros((B, 1), jnp.int32))


def _fps(xyz):
    return pl.pallas_call(
        _fps_body,
        out_shape=jax.ShapeDtypeStruct((B, 3, NQ), jnp.float32),
        scratch_shapes=[pltpu.VMEM((B, N), jnp.float32)],
        interpret=_INTERPRET,
    )(xyz)


_QT = 512


def _knn_body(nxyz_ref, xyz_ref, idx_ref, dist_ref):
    q = jnp.transpose(nxyz_ref[0])
    kk = xyz_ref[0]
    cross = lax.dot_general(q, kk, (((1,), (0,)), ((), ())),
                            preferred_element_type=jnp.float32)
    qsq = jnp.sum(q * q, axis=1, keepdims=True)
    ksq = jnp.sum(kk * kk, axis=0, keepdims=True)
    dist_ref[...] = (-2.0 * cross + qsq) + ksq
    lane = lax.broadcasted_iota(jnp.int32, (_QT, N), 1)
    krow = lax.broadcasted_iota(jnp.int32, (1, K, _QT), 1)

    def body(k, _):
        dd = dist_ref[...]
        am = jnp.argmin(dd, axis=1).astype(jnp.int32)
        idx_ref[...] = jnp.where(krow == k, am[None, None, :], idx_ref[...])
        dist_ref[...] = jnp.where(lane == am[:, None], jnp.inf, dd)
        return 0

    lax.fori_loop(0, K, body, 0)


def _knn(new_xyz, xyz):
    return pl.pallas_call(
        _knn_body,
        grid=(B, NQ // _QT),
        in_specs=[
            pl.BlockSpec((1, 3, _QT), lambda b, t: (b, 0, t)),
            pl.BlockSpec((1, 3, N), lambda b, t: (b, 0, 0)),
        ],
        out_specs=pl.BlockSpec((1, K, _QT), lambda b, t: (b, 0, t)),
        out_shape=jax.ShapeDtypeStruct((B, K, NQ), jnp.int32),
        scratch_shapes=[pltpu.VMEM((_QT, N), jnp.float32)],
        interpret=_INTERPRET,
    )(new_xyz, xyz)


def _prep_body(xyzT_ref, ptsT_ref, out_ref):
    xt = xyzT_ref[0]
    pad = jnp.zeros((N, 13), jnp.float32)
    out_ref[0, :, pl.ds(0, 16)] = jnp.concatenate([xt, pad], axis=1)
    out_ref[0, :, pl.ds(16, 32)] = ptsT_ref[0]


def _prep(xyzT, ptsT):
    return pl.pallas_call(
        _prep_body,
        grid=(B,),
        in_specs=[
            pl.BlockSpec((1, N, 3), lambda b: (b, 0, 0)),
            pl.BlockSpec((1, N, 32), lambda b: (b, 0, 0)),
        ],
        out_specs=pl.BlockSpec((1, N, 128), lambda b: (b, 0, 0)),
        out_shape=jax.ShapeDtypeStruct((B, N, 128), jnp.float32),
        interpret=_INTERPRET,
    )(xyzT, ptsT)


_P = B * NQ * K
_NW = 32
_CH = 512


def _sc_gather(table, gidx):
    mesh = plsc.VectorSubcoreMesh(core_axis_name="c", subcore_axis_name="s")
    per_w = _P // _NW

    @functools.partial(
        pl.kernel,
        mesh=mesh,
        out_type=jax.ShapeDtypeStruct((_P, 128), jnp.float32),
        scratch_types=[
            pltpu.VMEM((_CH,), jnp.int32),
            pltpu.VMEM((_CH, 128), jnp.float32),
            pltpu.SemaphoreType.DMA,
        ],
    )
    def k(table_hbm, idx_hbm, out_hbm, idx_v, rows_v, sem):
        wid = lax.axis_index("s") * 2 + lax.axis_index("c")
        base = wid * per_w

        def body(j, _):
            off = base + j * _CH
            pltpu.sync_copy(idx_hbm.at[pl.ds(off, _CH)], idx_v)
            pltpu.async_copy(table_hbm.at[idx_v], rows_v, sem).wait()
            pltpu.sync_copy(rows_v, out_hbm.at[pl.ds(off, _CH)])
            return 0

        lax.fori_loop(0, per_w // _CH, body, 0)

    return k(table, gidx)


_NT = 256
_G = B * NQ // _NT
_ROWS = _NT * K


def _y0(cg_ref, nxT_ref, W0pT_ref, b0_ref):
    a = cg_ref[0][:, :, :48]
    q = nxT_ref[0]
    qpad = jnp.concatenate([q, jnp.zeros((_NT, 45), jnp.float32)], axis=1)
    a = a - qpad[:, None, :]
    y = lax.dot_general(a.reshape(_ROWS, 48).astype(jnp.bfloat16), W0pT_ref[...],
                        (((1,), (0,)), ((), ())),
                        preferred_element_type=jnp.float32)
    return y + b0_ref[...]


def _bnrelu(y, s_ref, t_ref):
    return jnp.maximum(y * s_ref[...] + t_ref[...], 0.0)


def _mm(z, w_ref, b_ref):
    y = lax.dot_general(z.astype(jnp.bfloat16), w_ref[...], (((1,), (1,)), ((), ())),
                        preferred_element_type=jnp.float32)
    return y + b_ref[...]


def _stats_accum(y2d, out_ref):
    s = jnp.sum(y2d, axis=0, keepdims=True)
    ss = jnp.sum(y2d * y2d, axis=0, keepdims=True)
    both = jnp.concatenate([s, ss], axis=0)

    @pl.when(pl.program_id(0) == 0)
    def _():
        out_ref[...] = jnp.zeros_like(out_ref)

    out_ref[...] += both


_cg_spec = pl.BlockSpec((1, _NT, K, 128), lambda g: (g // 4, g % 4, 0, 0))
_nx_spec = pl.BlockSpec((1, _NT, 3), lambda g: (g // 4, g % 4, 0))


def _vec_spec(c):
    return pl.BlockSpec((1, c), lambda g: (0, 0))


def _w_spec(o, c):
    return pl.BlockSpec((o, c), lambda g: (0, 0))


def _pass1_body(cg_ref, nxT_ref, W0pT_ref, b0_ref, out_ref):
    y0 = _y0(cg_ref, nxT_ref, W0pT_ref, b0_ref)
    _stats_accum(y0, out_ref)


def _pass2_body(cg_ref, nxT_ref, W0pT_ref, b0_ref, s0_ref, t0_ref, W1_ref,
                b1_ref, out_ref):
    z0 = _bnrelu(_y0(cg_ref, nxT_ref, W0pT_ref, b0_ref), s0_ref, t0_ref)
    y1 = _mm(z0, W1_ref, b1_ref)
    _stats_accum(y1, out_ref)


def _pass3_body(cg_ref, nxT_ref, W0pT_ref, b0_ref, s0_ref, t0_ref, W1_ref,
                b1_ref, s1_ref, t1_ref, W2_ref, b2_ref, out_ref):
    z0 = _bnrelu(_y0(cg_ref, nxT_ref, W0pT_ref, b0_ref), s0_ref, t0_ref)
    z1 = _bnrelu(_mm(z0, W1_ref, b1_ref), s1_ref, t1_ref)
    y2 = _mm(z1, W2_ref, b2_ref)
    _stats_accum(y2, out_ref)


def _pass4_body(cg_ref, nxT_ref, W0pT_ref, b0_ref, s0_ref, t0_ref, W1_ref,
                b1_ref, s1_ref, t1_ref, W2_ref, b2_ref, s2_ref, t2_ref, out_ref):
    z0 = _bnrelu(_y0(cg_ref, nxT_ref, W0pT_ref, b0_ref), s0_ref, t0_ref)
    z1 = _bnrelu(_mm(z0, W1_ref, b1_ref), s1_ref, t1_ref)
    z2 = _bnrelu(_mm(z1, W2_ref, b2_ref), s2_ref, t2_ref)
    out_ref[0] = jnp.max(z2.reshape(_NT, K, 64), axis=1)


def _stats_call(body, extra_specs, cout, args):
    return pl.pallas_call(
        body,
        grid=(_G,),
        in_specs=[_cg_spec, _nx_spec, _w_spec(48, 32), _vec_spec(32)] + extra_specs,
        out_specs=pl.BlockSpec((2, cout), lambda g: (0, 0)),
        out_shape=jax.ShapeDtypeStruct((2, cout), jnp.float32),
        interpret=_INTERPRET,
    )(*args)


def _finalize(stats, gamma, beta):
    mean = stats[0] / _P
    var = stats[1] / _P - mean * mean
    s = gamma / jnp.sqrt(var + EPS)
    t = beta - mean * s
    return s[None, :], t[None, :]


def kernel(xyz, points, W0, b0, g0, be0, W1, b1, g1, be1, W2, b2, g2, be2):
    xyzT = jnp.transpose(xyz, (0, 2, 1))
    ptsT = jnp.transpose(points, (0, 2, 1))
    W0pT = jnp.concatenate(
        [jnp.transpose(W0[:, :3]), jnp.zeros((13, 32), jnp.float32),
         jnp.transpose(W0[:, 3:])], axis=0).astype(jnp.bfloat16)
    W1b = W1.astype(jnp.bfloat16)
    W2b = W2.astype(jnp.bfloat16)

    new_xyz = _fps(xyz)
    idx = _knn(new_xyz, xyz)
    table = _prep(xyzT, ptsT)

    gidx = (jnp.transpose(idx, (0, 2, 1))
            + (jnp.arange(B, dtype=jnp.int32) * N)[:, None, None]).reshape(-1)
    cg = _sc_gather(table.reshape(B * N, 128), gidx)
    cg4 = cg.reshape(B, NQ, K, 128)
    nxT = jnp.transpose(new_xyz, (0, 2, 1))

    b0r, b1r, b2r = b0[None, :], b1[None, :], b2[None, :]
    st0 = _stats_call(_pass1_body, [], 32, (cg4, nxT, W0pT, b0r))
    s0, t0 = _finalize(st0, g0, be0)
    st1 = _stats_call(_pass2_body,
                      [_vec_spec(32), _vec_spec(32), _w_spec(32, 32),
                       _vec_spec(32)], 32,
                      (cg4, nxT, W0pT, b0r, s0, t0, W1b, b1r))
    s1, t1 = _finalize(st1, g1, be1)
    st2 = _stats_call(_pass3_body,
                      [_vec_spec(32), _vec_spec(32), _w_spec(32, 32),
                       _vec_spec(32), _vec_spec(32), _vec_spec(32),
                       _w_spec(64, 32), _vec_spec(64)], 64,
                      (cg4, nxT, W0pT, b0r, s0, t0, W1b, b1r, s1, t1, W2b, b2r))
    s2, t2 = _finalize(st2, g2, be2)

    out = pl.pallas_call(
        _pass4_body,
        grid=(_G,),
        in_specs=[_cg_spec, _nx_spec, _w_spec(48, 32), _vec_spec(32),
                  _vec_spec(32), _vec_spec(32), _w_spec(32, 32), _vec_spec(32),
                  _vec_spec(32), _vec_spec(32), _w_spec(64, 32), _vec_spec(64),
                  _vec_spec(64), _vec_spec(64)],
        out_specs=pl.BlockSpec((1, _NT, 64), lambda g: (g // 4, g % 4, 0)),
        out_shape=jax.ShapeDtypeStruct((B, NQ, 64), jnp.float32),
        interpret=_INTERPRET,
    )(cg4, nxT, W0pT, b0r, s0, t0, W1b, b1r, s1, t1, W2b, b2r, s2, t2)

    return (new_xyz, jnp.transpose(out, (0, 2, 1)))
